# baseline (speedup 1.0000x reference)
import functools

import jax
import jax.numpy as jnp
from jax.experimental import pallas as pl
from jax.experimental.pallas import tpu as pltpu


def _moe_block_kernel(x_ref, xb_ref, wr_ref, br_ref, w1_ref, b1_ref, w2_ref,
                      b2_ref, out_ref, w_scratch):
    e = pl.program_id(1)

    @pl.when(e == 0)
    def _router():
        s = jnp.dot(x_ref[...], wr_ref[...], preferred_element_type=jnp.float32)
        s = s + br_ref[...]
        m1 = jnp.max(s, axis=-1, keepdims=True)
        is_max = s == m1
        iota = jax.lax.broadcasted_iota(jnp.int32, s.shape, 1)
        i1 = jnp.min(jnp.where(is_max, iota, s.shape[-1]), axis=-1, keepdims=True)
        first_max = iota == i1
        m2 = jnp.max(jnp.where(first_max, -jnp.inf, s), axis=-1, keepdims=True)
        mask = jnp.where(s >= m2, s, 0.0)
        mx = jnp.max(mask, axis=-1, keepdims=True)
        ex = jnp.exp(mask - mx)
        w = ex / jnp.sum(ex, axis=-1, keepdims=True)
        w_scratch[...] = w
        out_ref[...] = jnp.dot(w, b2_ref[...], preferred_element_type=jnp.float32)

    h = jnp.dot(xb_ref[...], w1_ref[0], preferred_element_type=jnp.float32)
    h = jnp.maximum(h + b1_ref[0, 0], 0.0).astype(jnp.bfloat16)
    o = jnp.dot(h, w2_ref[0], preferred_element_type=jnp.float32)
    w = w_scratch[...]
    onehot = (jax.lax.broadcasted_iota(jnp.int32, w.shape, 1) == e)
    w_e = jnp.sum(jnp.where(onehot, w, 0.0), axis=-1, keepdims=True)
    out_ref[...] += w_e * o


@functools.partial(jax.jit, static_argnames=("block_m",))
def _moe(x, W1, b1, W2, b2, Wr, br, block_m=512):
    BT, D = x.shape
    E, _, H = W1.shape
    grid = (BT // block_m, E)
    out = pl.pallas_call(
        _moe_block_kernel,
        grid=grid,
        in_specs=[
            pl.BlockSpec((block_m, D), lambda t, e: (t, 0)),
            pl.BlockSpec((block_m, D), lambda t, e: (t, 0)),
            pl.BlockSpec((D, E), lambda t, e: (0, 0)),
            pl.BlockSpec((1, E), lambda t, e: (0, 0)),
            pl.BlockSpec((1, D, H), lambda t, e: (e, 0, 0)),
            pl.BlockSpec((1, 1, H), lambda t, e: (e, 0, 0)),
            pl.BlockSpec((1, H, D), lambda t, e: (e, 0, 0)),
            pl.BlockSpec((E, D), lambda t, e: (0, 0)),
        ],
        out_specs=pl.BlockSpec((block_m, D), lambda t, e: (t, 0)),
        out_shape=jax.ShapeDtypeStruct((BT, D), jnp.float32),
        scratch_shapes=[pltpu.VMEM((block_m, E), jnp.float32)],
        compiler_params=pltpu.CompilerParams(
            dimension_semantics=("parallel", "arbitrary"),
        ),
    )(x, x.astype(jnp.bfloat16), Wr, br.reshape(1, E),
      W1.astype(jnp.bfloat16), b1.reshape(E, 1, H),
      W2.astype(jnp.bfloat16), b2)
    return out


def kernel(inputs, W1, b1, W2, b2, Wr, br):
    B, T, D = inputs.shape
    x = inputs.reshape(B * T, D)
    out = _moe(x, W1, b1, W2, b2, Wr, br)
    return out.reshape(B, T, D)


# speedup vs baseline: 1.0737x; 1.0737x over previous
import functools

import jax
import jax.numpy as jnp
from jax.experimental import pallas as pl
from jax.experimental.pallas import tpu as pltpu


def _moe_block_kernel(x_ref, wr_ref, br_ref, w1_ref, b1_ref, w2_ref,
                      b2_ref, out_ref, w_scratch):
    e = pl.program_id(1)

    @pl.when(e == 0)
    def _router():
        s = jnp.dot(x_ref[...], wr_ref[...], preferred_element_type=jnp.float32)
        s = s + br_ref[...]
        m1 = jnp.max(s, axis=-1, keepdims=True)
        is_max = s == m1
        iota = jax.lax.broadcasted_iota(jnp.int32, s.shape, 1)
        i1 = jnp.min(jnp.where(is_max, iota, s.shape[-1]), axis=-1, keepdims=True)
        first_max = iota == i1
        m2 = jnp.max(jnp.where(first_max, -jnp.inf, s), axis=-1, keepdims=True)
        mask = jnp.where(s >= m2, s, 0.0)
        mx = jnp.max(mask, axis=-1, keepdims=True)
        ex = jnp.exp(mask - mx)
        w = ex / jnp.sum(ex, axis=-1, keepdims=True)
        w_scratch[...] = w
        out_ref[...] = jnp.dot(w, b2_ref[...], preferred_element_type=jnp.float32)

    h = jnp.dot(x_ref[...], w1_ref[0], preferred_element_type=jnp.float32)
    h = jnp.maximum(h + b1_ref[0, 0], 0.0)
    o = jnp.dot(h, w2_ref[0], preferred_element_type=jnp.float32)
    w = w_scratch[...]
    onehot = (jax.lax.broadcasted_iota(jnp.int32, w.shape, 1) == e)
    w_e = jnp.sum(jnp.where(onehot, w, 0.0), axis=-1, keepdims=True)
    out_ref[...] += w_e * o


@functools.partial(jax.jit, static_argnames=("block_m",))
def _moe(x, W1, b1, W2, b2, Wr, br, block_m=512):
    BT, D = x.shape
    E, _, H = W1.shape
    grid = (BT // block_m, E)
    out = pl.pallas_call(
        _moe_block_kernel,
        grid=grid,
        in_specs=[
            pl.BlockSpec((block_m, D), lambda t, e: (t, 0)),
            pl.BlockSpec((D, E), lambda t, e: (0, 0)),
            pl.BlockSpec((1, E), lambda t, e: (0, 0)),
            pl.BlockSpec((1, D, H), lambda t, e: (e, 0, 0)),
            pl.BlockSpec((1, 1, H), lambda t, e: (e, 0, 0)),
            pl.BlockSpec((1, H, D), lambda t, e: (e, 0, 0)),
            pl.BlockSpec((E, D), lambda t, e: (0, 0)),
        ],
        out_specs=pl.BlockSpec((block_m, D), lambda t, e: (t, 0)),
        out_shape=jax.ShapeDtypeStruct((BT, D), jnp.float32),
        scratch_shapes=[pltpu.VMEM((block_m, E), jnp.float32)],
        compiler_params=pltpu.CompilerParams(
            dimension_semantics=("parallel", "arbitrary"),
        ),
    )(x, Wr, br.reshape(1, E), W1, b1.reshape(E, 1, H), W2, b2)
    return out


def kernel(inputs, W1, b1, W2, b2, Wr, br):
    B, T, D = inputs.shape
    x = inputs.reshape(B * T, D)
    out = _moe(x, W1, b1, W2, b2, Wr, br)
    return out.reshape(B, T, D)


# speedup vs baseline: 1.2259x; 1.1418x over previous
import functools

import jax
import jax.numpy as jnp
from jax.experimental import pallas as pl
from jax.experimental.pallas import tpu as pltpu


def _moe_grid_kernel(x_ref, wr_ref, br_ref, w1_ref, b1_ref, w2_ref, b2_ref,
                     out_ref, w_scratch):
    e = pl.program_id(0)
    hb = pl.program_id(1)

    @pl.when((e == 0) & (hb == 0))
    def _router():
        s = jnp.dot(x_ref[...], wr_ref[...], preferred_element_type=jnp.float32)
        s = s + br_ref[...]
        m1 = jnp.max(s, axis=-1, keepdims=True)
        is_max = s == m1
        iota = jax.lax.broadcasted_iota(jnp.int32, s.shape, 1)
        i1 = jnp.min(jnp.where(is_max, iota, s.shape[-1]), axis=-1, keepdims=True)
        m2 = jnp.max(jnp.where(iota == i1, -jnp.inf, s), axis=-1, keepdims=True)
        mask = jnp.where(s >= m2, s, 0.0)
        mx = jnp.max(mask, axis=-1, keepdims=True)
        ex = jnp.exp(mask - mx)
        w = ex / jnp.sum(ex, axis=-1, keepdims=True)
        w_scratch[...] = w
        out_ref[...] = jnp.dot(w, b2_ref[...], preferred_element_type=jnp.float32)

    h = jnp.dot(x_ref[...], w1_ref[0], preferred_element_type=jnp.float32)
    h = jnp.maximum(h + b1_ref[0, 0], 0.0)
    o = jnp.dot(h, w2_ref[0], preferred_element_type=jnp.float32)
    w = w_scratch[...]
    onehot = (jax.lax.broadcasted_iota(jnp.int32, w.shape, 1) == e)
    w_e = jnp.sum(jnp.where(onehot, w, 0.0), axis=-1, keepdims=True)
    out_ref[...] += w_e * o


@functools.partial(jax.jit, static_argnames=("block_h",))
def _moe(x, W1, b1, W2, b2, Wr, br, block_h=768):
    BT, D = x.shape
    E, _, H = W1.shape
    grid = (E, H // block_h)
    out = pl.pallas_call(
        _moe_grid_kernel,
        grid=grid,
        in_specs=[
            pl.BlockSpec((BT, D), lambda e, hb: (0, 0)),
            pl.BlockSpec((D, E), lambda e, hb: (0, 0)),
            pl.BlockSpec((1, E), lambda e, hb: (0, 0)),
            pl.BlockSpec((1, D, block_h), lambda e, hb: (e, 0, hb)),
            pl.BlockSpec((1, 1, block_h), lambda e, hb: (e, 0, hb)),
            pl.BlockSpec((1, block_h, D), lambda e, hb: (e, hb, 0)),
            pl.BlockSpec((E, D), lambda e, hb: (0, 0)),
        ],
        out_specs=pl.BlockSpec((BT, D), lambda e, hb: (0, 0)),
        out_shape=jax.ShapeDtypeStruct((BT, D), jnp.float32),
        scratch_shapes=[pltpu.VMEM((BT, E), jnp.float32)],
        compiler_params=pltpu.CompilerParams(
            dimension_semantics=("arbitrary", "arbitrary"),
        ),
    )(x, Wr, br.reshape(1, E), W1, b1.reshape(E, 1, H), W2, b2)
    return out


def kernel(inputs, W1, b1, W2, b2, Wr, br):
    B, T, D = inputs.shape
    x = inputs.reshape(B * T, D)
    out = _moe(x, W1, b1, W2, b2, Wr, br)
    return out.reshape(B, T, D)
